# Initial kernel scaffold; baseline (speedup 1.0000x reference)
import jax
import jax.numpy as jnp
from jax.experimental import pallas as pl


def kernel(x, embed_table):
    raise NotImplementedError("write your pallas kernel here")


# speedup vs baseline: 4.8591x; 4.8591x over previous
import jax
import jax.numpy as jnp
from jax.experimental import pallas as pl
from jax.experimental.pallas import tpu as pltpu


def kernel(x, embed_table):
    b, _, h, w = x.shape
    hw = h * w
    d = embed_table.shape[1]

    bb = 8
    grid = b // bb

    def body(e_ref, o_ref, scratch):
        @pl.when(pl.program_id(0) == 0)
        def _():
            scratch[...] = e_ref[...].T

        o_ref[...] = jnp.broadcast_to(scratch[...][None], (bb, d, hw))

    out = pl.pallas_call(
        body,
        grid=(grid,),
        in_specs=[pl.BlockSpec((hw, d), lambda i: (0, 0))],
        out_specs=pl.BlockSpec((bb, d, hw), lambda i: (i, 0, 0)),
        out_shape=jax.ShapeDtypeStruct((b, d, hw), embed_table.dtype),
        scratch_shapes=[pltpu.VMEM((d, hw), embed_table.dtype)],
    )(embed_table)
    return out.reshape(b, d, h, w)
